# baseline (speedup 1.0000x reference)
import functools
import jax
import jax.numpy as jnp
from jax import lax
from jax.experimental import pallas as pl
from jax.experimental.pallas import tpu as pltpu
from jax.experimental.pallas import tpu_sc as plsc

T = 2048
D = 1024
H = 16
KVH = 4
HD = 64
E = 8
TOPK = 2
I = 768
THETA = 1000000.0
EPS = 1e-6

BT = 256
BK = 512
EPAD = 128
NEG = jnp.finfo(jnp.float32).min


def _norm_rope(xall, nheads, cs64, sn64s, wn_tiled):
    w = nheads * HD
    hm = (jax.lax.broadcasted_iota(jnp.int32, (w, nheads), 0) // HD
          == jax.lax.broadcasted_iota(jnp.int32, (w, nheads), 1)).astype(jnp.float32)
    hmt = (jax.lax.broadcasted_iota(jnp.int32, (nheads, w), 0)
           == jax.lax.broadcasted_iota(jnp.int32, (nheads, w), 1) // HD).astype(jnp.float32)
    ssq = jnp.dot(xall * xall, hm, preferred_element_type=jnp.float32)
    invn = jax.lax.rsqrt(ssq * (1.0 / HD) + EPS)
    invb = jnp.dot(invn, hmt, preferred_element_type=jnp.float32)
    xn = xall * invb * wn_tiled
    lane = jax.lax.broadcasted_iota(jnp.int32, (BT, w), 1)
    first_half = (lane % HD) < (HD // 2)
    rot = jnp.where(first_half,
                    pltpu.roll(xn, w - HD // 2, axis=1),
                    pltpu.roll(xn, HD // 2, axis=1))
    csf = jnp.concatenate([cs64] * nheads, axis=1)
    snf = jnp.concatenate([sn64s] * nheads, axis=1)
    return xn * csf + rot * snf


def _pre_kernel(pos_ref, x_ref, w_ref, ln1_ref, qn_ref, kn_ref, q_ref, k_ref, v_ref):
    x = x_ref[...]
    h = x * jax.lax.rsqrt(jnp.mean(x * x, axis=-1, keepdims=True) + EPS) * ln1_ref[...]
    qkv = jnp.dot(h.astype(jnp.bfloat16), w_ref[...], preferred_element_type=jnp.float32)
    pos = pos_ref[...].astype(jnp.float32)
    half_iota = jax.lax.broadcasted_iota(jnp.int32, (1, HD // 2), 1).astype(jnp.float32)
    inv_freq = jnp.exp(half_iota * (-2.0 / HD) * jnp.log(THETA))
    freqs = pos * inv_freq
    cs = jnp.cos(freqs)
    sn = jnp.sin(freqs)
    cs64 = jnp.concatenate([cs, cs], axis=1)
    sn64s = jnp.concatenate([-sn, sn], axis=1)
    q_rope = _norm_rope(qkv[:, :H * HD], H, cs64, sn64s, qn_ref[...]).astype(jnp.bfloat16)
    k_rope = _norm_rope(qkv[:, H * HD:(H + KVH) * HD], KVH, cs64, sn64s,
                        kn_ref[...]).astype(jnp.bfloat16)
    for hh in range(H):
        q_ref[hh] = q_rope[:, hh * HD:(hh + 1) * HD]
    for hh in range(KVH):
        k_ref[hh] = k_rope[:, hh * HD:(hh + 1) * HD]
        v_ref[hh] = qkv[:, (H + KVH) * HD + hh * HD:
                        (H + KVH) * HD + (hh + 1) * HD].astype(jnp.bfloat16)


REP = H // KVH


def _flash_kernel(q_ref, k_ref, v_ref, o_ref):
    tq = pl.program_id(1)
    qs = [q_ref[i] * jnp.bfloat16(HD ** -0.5) for i in range(REP)]

    def chunk(j, ms, ls, accs, masked):
        k = k_ref[0, pl.ds(j * BK, BK), :]
        v = v_ref[0, pl.ds(j * BK, BK), :]
        if masked:
            rows = tq * BT + jax.lax.broadcasted_iota(jnp.int32, (BT, BK), 0)
            cols = j * BK + jax.lax.broadcasted_iota(jnp.int32, (BT, BK), 1)
            keep = rows >= cols
        nms, nls, naccs = [], [], []
        for i in range(REP):
            s = jax.lax.dot_general(qs[i], k, (((1,), (1,)), ((), ())),
                                    preferred_element_type=jnp.float32)
            if masked:
                s = jnp.where(keep, s, NEG)
            m_new = jnp.maximum(ms[i], jnp.max(s, axis=-1, keepdims=True))
            p = jnp.exp(s - m_new)
            alpha = jnp.exp(ms[i] - m_new)
            nls.append(ls[i] * alpha + jnp.sum(p, axis=-1, keepdims=True))
            naccs.append(accs[i] * alpha
                         + jnp.dot(p.astype(jnp.bfloat16), v,
                                   preferred_element_type=jnp.float32))
            nms.append(m_new)
        return nms, nls, naccs

    m0 = [jnp.full((BT, 1), NEG, jnp.float32)] * REP
    l0 = [jnp.zeros((BT, 1), jnp.float32)] * REP
    a0 = [jnp.zeros((BT, HD), jnp.float32)] * REP
    nfull = (tq * BT) // BK
    ms, ls, accs = jax.lax.fori_loop(
        0, nfull, lambda j, c: chunk(j, *c, masked=False), (m0, l0, a0))
    ms, ls, accs = chunk(nfull, ms, ls, accs, masked=True)
    for i in range(REP):
        o_ref[i] = (accs[i] / ls[i]).astype(jnp.bfloat16)


def _post_kernel(o_ref, res_ref, wo_ref, ln2_ref, gate_ref, h1_ref, h2_ref, w_ref):
    o_all = jnp.concatenate([o_ref[hh] for hh in range(H)], axis=1)
    attn = jnp.dot(o_all, wo_ref[...], preferred_element_type=jnp.float32)
    h1 = res_ref[...] + attn
    h1_ref[...] = h1
    h2 = h1 * jax.lax.rsqrt(jnp.mean(h1 * h1, axis=-1, keepdims=True) + EPS) * ln2_ref[...]
    h2_ref[...] = h2
    logits = jnp.dot(h2, gate_ref[...], preferred_element_type=jnp.float32)
    col = jax.lax.broadcasted_iota(jnp.int32, (BT, EPAD), 1)
    valid = col < E
    lm = jnp.where(valid, logits, NEG)
    mx = jnp.max(lm, axis=-1, keepdims=True)
    p = jnp.where(valid, jnp.exp(lm - mx), 0.0)
    rw = p / jnp.sum(p, axis=-1, keepdims=True)
    m1 = jnp.max(rw, axis=-1, keepdims=True)
    i1 = jnp.min(jnp.where(rw == m1, col, EPAD), axis=-1, keepdims=True)
    f1 = col == i1
    rw2 = jnp.where(f1, -1.0, rw)
    m2 = jnp.max(rw2, axis=-1, keepdims=True)
    i2 = jnp.min(jnp.where(rw2 == m2, col, EPAD), axis=-1, keepdims=True)
    f2 = col == i2
    denom = m1 + m2
    w = (jnp.where(f1, m1, 0.0) + jnp.where(f2, m2, 0.0)) / denom
    w_ref[...] = w


BLKM = 128
P = 5120
NBLK = P // BLKM


def _sched_kernel(w_ref, dest_ref, w8_ref, be_ref, d0_ref, d1_ref):
    w = w_ref[...]
    maskf = (w > 0.0).astype(jnp.float32)
    ri = jax.lax.broadcasted_iota(jnp.int32, (BT, BT), 0)
    ci = jax.lax.broadcasted_iota(jnp.int32, (BT, BT), 1)
    lstrict = (ri > ci).astype(jnp.float32)
    base = jnp.zeros((1, EPAD), jnp.float32)
    ranks = []
    for c in range(T // BT):
        seg = maskf[c * BT:(c + 1) * BT]
        within = jnp.dot(lstrict, seg, preferred_element_type=jnp.float32)
        ranks.append(within + base)
        base = base + jnp.sum(seg, axis=0, keepdims=True)
    rank = jnp.concatenate(ranks, axis=0)
    counts = base
    pc = jnp.ceil(counts * (1.0 / BLKM)) * BLKM
    ri2 = jax.lax.broadcasted_iota(jnp.int32, (EPAD, EPAD), 0)
    ci2 = jax.lax.broadcasted_iota(jnp.int32, (EPAD, EPAD), 1)
    ustrict = (ri2 < ci2).astype(jnp.float32)
    starts = jnp.dot(pc, ustrict, preferred_element_type=jnp.float32)
    dest = starts + rank
    dest_ref[...] = dest.astype(jnp.int32)
    w8_ref[...] = w
    bvals = (jax.lax.broadcasted_iota(jnp.int32, (EPAD, EPAD), 0) * BLKM).astype(jnp.float32)
    m = (jnp.broadcast_to(starts, (EPAD, EPAD)) <= bvals).astype(jnp.float32)
    be = jnp.sum(m, axis=1, keepdims=True) - 1.0
    be_ref[...] = jnp.minimum(be, float(E - 1)).astype(jnp.int32)
    mask = w > 0.0
    big = float(P - 1)
    d0 = jnp.min(jnp.where(mask, dest, big), axis=1, keepdims=True)
    d1 = jnp.min(jnp.where(mask & (dest > d0), dest, big), axis=1, keepdims=True)
    d0_ref[...] = d0.astype(jnp.int32)
    d1_ref[...] = d1.astype(jnp.int32)


def _group_kernel(be_ref, xs_ref, wgu_ref, wd_ref, ww_ref, out_ref):
    x = xs_ref[...].astype(jnp.bfloat16)
    gu = jnp.dot(x, wgu_ref[0], preferred_element_type=jnp.float32)
    g = gu[:, :I]
    u = gu[:, I:]
    act = (g / (1.0 + jnp.exp(-g))) * u * ww_ref[...]
    out_ref[...] = jnp.dot(act.astype(jnp.bfloat16), wd_ref[0],
                           preferred_element_type=jnp.float32)


def _final_kernel(h1_ref, g0_ref, g1_ref, out_ref):
    out_ref[...] = h1_ref[...] + g0_ref[...] + g1_ref[...]


SC_NC = 2
SC_NS = 16
SC_NW = SC_NC * SC_NS
LANES = 16


@functools.cache
def _sc_dispatch():
    @functools.partial(
        pl.kernel,
        mesh=plsc.VectorSubcoreMesh(core_axis_name="c", subcore_axis_name="s"),
        compiler_params=pltpu.CompilerParams(needs_layout_passes=False),
        out_type=jax.ShapeDtypeStruct((P,), jnp.float32),
        scratch_types=[
            pltpu.VMEM((T * E,), jnp.int32),
            pltpu.VMEM((T * E,), jnp.float32),
            pltpu.VMEM((P,), jnp.float32),
        ],
    )
    def dispatch(dest_hbm, w8_hbm, ww_out, dest_v, wv_v, ww_v):
        cid = lax.axis_index("c")
        sid = lax.axis_index("s")

        @pl.when((cid == 0) & (sid == 0))
        def _():
            pltpu.sync_copy(dest_hbm, dest_v)
            pltpu.sync_copy(w8_hbm, wv_v)

            def init(i, carry):
                ww_v[pl.ds(i * LANES, LANES)] = jnp.zeros((LANES,), jnp.float32)
                return carry

            lax.fori_loop(0, P // LANES, init, 0)

            def body(i, carry):
                d = dest_v[pl.ds(i * LANES, LANES)]
                wv = wv_v[pl.ds(i * LANES, LANES)]
                m = wv > 0.0
                plsc.store_scatter(ww_v, [d], wv, mask=m)
                return carry

            lax.fori_loop(0, (T * E) // LANES, body, 0)
            pltpu.sync_copy(ww_v, ww_out)

    return dispatch


SCH = T // SC_NW


@functools.cache
def _sc_scatter_x():
    @functools.partial(
        pl.kernel,
        mesh=plsc.VectorSubcoreMesh(core_axis_name="c", subcore_axis_name="s"),
        compiler_params=pltpu.CompilerParams(needs_layout_passes=False),
        out_type=jax.ShapeDtypeStruct((P, D), jnp.float32),
        scratch_types=[
            pltpu.VMEM((SCH,), jnp.int32),
            pltpu.VMEM((SCH,), jnp.int32),
            pltpu.VMEM((SCH, D), jnp.float32),
            pltpu.SemaphoreType.DMA,
            pltpu.SemaphoreType.DMA,
        ],
    )
    def scatter_x(h2_hbm, d0_hbm, d1_hbm, xs_out, idx0, idx1, rows_v, s0, s1):
        wid = lax.axis_index("s") * SC_NC + lax.axis_index("c")
        base = wid * SCH
        pltpu.sync_copy(h2_hbm.at[pl.ds(base, SCH)], rows_v)
        pltpu.sync_copy(d0_hbm.at[pl.ds(base, SCH)], idx0)
        pltpu.sync_copy(d1_hbm.at[pl.ds(base, SCH)], idx1)
        c0 = pltpu.async_copy(rows_v, xs_out.at[idx0], s0)
        c1 = pltpu.async_copy(rows_v, xs_out.at[idx1], s1)
        c0.wait()
        c1.wait()

    return scatter_x


CCH = 32


@functools.cache
def _sc_combine_gather():
    @functools.partial(
        pl.kernel,
        mesh=plsc.VectorSubcoreMesh(core_axis_name="c", subcore_axis_name="s"),
        compiler_params=pltpu.CompilerParams(needs_layout_passes=False),
        out_type=[
            jax.ShapeDtypeStruct((T, D), jnp.float32),
            jax.ShapeDtypeStruct((T, D), jnp.float32),
        ],
        scratch_types=[
            pltpu.VMEM((CCH,), jnp.int32),
            pltpu.VMEM((CCH,), jnp.int32),
            pltpu.VMEM((CCH, D), jnp.float32),
            pltpu.VMEM((CCH, D), jnp.float32),
            pltpu.SemaphoreType.DMA,
            pltpu.SemaphoreType.DMA,
            pltpu.SemaphoreType.DMA,
            pltpu.SemaphoreType.DMA,
        ],
    )
    def combine(d0_hbm, d1_hbm, outs_hbm, g0_out, g1_out, idx0, idx1,
                rows0, rows1, gs0, gs1, os0, os1):
        wid = lax.axis_index("s") * SC_NC + lax.axis_index("c")
        base0 = wid * (T // SC_NW)
        nc = T // SC_NW // CCH
        jobs = []
        for c in range(nc):
            jobs.append((d0_hbm, g0_out, base0 + c * CCH))
            jobs.append((d1_hbm, g1_out, base0 + c * CCH))
        idx = [idx0, idx1]
        rows = [rows0, rows1]
        gs = [gs0, gs1]
        os_ = [os0, os1]
        cps = [None, None]
        ocs = [None, None]
        dsts = [None, None]
        for j, (src_idx, dst, base) in enumerate(jobs):
            b = j & 1
            if ocs[b] is not None:
                ocs[b].wait()
                ocs[b] = None
            pltpu.sync_copy(src_idx.at[pl.ds(base, CCH)], idx[b])
            cps[b] = pltpu.async_copy(outs_hbm.at[idx[b]], rows[b], gs[b])
            dsts[b] = (dst, base)
            ob = 1 - b
            if cps[ob] is not None:
                cps[ob].wait()
                cps[ob] = None
                pdst, pbase = dsts[ob]
                ocs[ob] = pltpu.async_copy(
                    rows[ob], pdst.at[pl.ds(pbase, CCH)], os_[ob])
        lb = (len(jobs) - 1) & 1
        cps[lb].wait()
        pdst, pbase = dsts[lb]
        ocs[lb] = pltpu.async_copy(rows[lb], pdst.at[pl.ds(pbase, CCH)], os_[lb])
        for b in (0, 1):
            if ocs[b] is not None:
                ocs[b].wait()

    return combine


def kernel(hidden_states, positions, W_qkv, q_norm_w, k_norm_w, W_o, ln1_w, ln2_w,
           gate_w, W_gate_up, W_down):
    pos2 = positions.reshape(T, 1)
    ln1 = ln1_w.reshape(1, D)
    ln2 = ln2_w.reshape(1, D)
    qn = jnp.tile(q_norm_w, H).reshape(1, H * HD)
    kn = jnp.tile(k_norm_w, KVH).reshape(1, KVH * HD)
    gate_pad = jnp.concatenate([gate_w, jnp.zeros((D, EPAD - E), jnp.float32)], axis=1)

    nt = T // BT
    q, k, v = pl.pallas_call(
        _pre_kernel,
        grid=(nt,),
        in_specs=[
            pl.BlockSpec((BT, 1), lambda t: (t, 0)),
            pl.BlockSpec((BT, D), lambda t: (t, 0)),
            pl.BlockSpec((D, (H + 2 * KVH) * HD), lambda t: (0, 0)),
            pl.BlockSpec((1, D), lambda t: (0, 0)),
            pl.BlockSpec((1, H * HD), lambda t: (0, 0)),
            pl.BlockSpec((1, KVH * HD), lambda t: (0, 0)),
        ],
        out_specs=[
            pl.BlockSpec((H, BT, HD), lambda t: (0, t, 0)),
            pl.BlockSpec((KVH, BT, HD), lambda t: (0, t, 0)),
            pl.BlockSpec((KVH, BT, HD), lambda t: (0, t, 0)),
        ],
        out_shape=[
            jax.ShapeDtypeStruct((H, T, HD), jnp.bfloat16),
            jax.ShapeDtypeStruct((KVH, T, HD), jnp.bfloat16),
            jax.ShapeDtypeStruct((KVH, T, HD), jnp.bfloat16),
        ],
    )(pos2, hidden_states, W_qkv.astype(jnp.bfloat16), ln1, qn, kn)

    o = pl.pallas_call(
        _flash_kernel,
        grid=(KVH, nt),
        in_specs=[
            pl.BlockSpec((REP, BT, HD), lambda g, t: (g, t, 0)),
            pl.BlockSpec((1, T, HD), lambda g, t: (g, 0, 0)),
            pl.BlockSpec((1, T, HD), lambda g, t: (g, 0, 0)),
        ],
        out_specs=pl.BlockSpec((REP, BT, HD), lambda g, t: (g, t, 0)),
        out_shape=jax.ShapeDtypeStruct((H, T, HD), jnp.bfloat16),
    )(q, k, v)

    h1, h2, w = pl.pallas_call(
        _post_kernel,
        grid=(nt,),
        in_specs=[
            pl.BlockSpec((H, BT, HD), lambda t: (0, t, 0)),
            pl.BlockSpec((BT, D), lambda t: (t, 0)),
            pl.BlockSpec((H * HD, D), lambda t: (0, 0)),
            pl.BlockSpec((1, D), lambda t: (0, 0)),
            pl.BlockSpec((D, EPAD), lambda t: (0, 0)),
        ],
        out_specs=[
            pl.BlockSpec((BT, D), lambda t: (t, 0)),
            pl.BlockSpec((BT, D), lambda t: (t, 0)),
            pl.BlockSpec((BT, EPAD), lambda t: (t, 0)),
        ],
        out_shape=[
            jax.ShapeDtypeStruct((T, D), jnp.float32),
            jax.ShapeDtypeStruct((T, D), jnp.float32),
            jax.ShapeDtypeStruct((T, EPAD), jnp.float32),
        ],
    )(o, hidden_states, W_o.astype(jnp.bfloat16), ln2, gate_pad)

    dest, w8, be2, d0, d1 = pl.pallas_call(
        _sched_kernel,
        grid=(1,),
        in_specs=[pl.BlockSpec((T, EPAD), lambda i: (0, 0))],
        out_specs=[
            pl.BlockSpec((T, EPAD), lambda i: (0, 0)),
            pl.BlockSpec((T, EPAD), lambda i: (0, 0)),
            pl.BlockSpec((EPAD, 1), lambda i: (0, 0)),
            pl.BlockSpec((T, 1), lambda i: (0, 0)),
            pl.BlockSpec((T, 1), lambda i: (0, 0)),
        ],
        out_shape=[
            jax.ShapeDtypeStruct((T, EPAD), jnp.int32),
            jax.ShapeDtypeStruct((T, EPAD), jnp.float32),
            jax.ShapeDtypeStruct((EPAD, 1), jnp.int32),
            jax.ShapeDtypeStruct((T, 1), jnp.int32),
            jax.ShapeDtypeStruct((T, 1), jnp.int32),
        ],
    )(w)
    be = be2.reshape(EPAD)[:NBLK]

    dest8 = dest[:, :E].reshape(T * E)
    w8f = w8[:, :E].reshape(T * E)
    row_weight = _sc_dispatch()(dest8, w8f)

    xs = _sc_scatter_x()(h2, d0.reshape(T), d1.reshape(T))

    outs = pl.pallas_call(
        _group_kernel,
        grid_spec=pltpu.PrefetchScalarGridSpec(
            num_scalar_prefetch=1,
            grid=(NBLK,),
            in_specs=[
                pl.BlockSpec((BLKM, D), lambda b, be_r: (b, 0)),
                pl.BlockSpec((1, D, 2 * I), lambda b, be_r: (be_r[b], 0, 0)),
                pl.BlockSpec((1, I, D), lambda b, be_r: (be_r[b], 0, 0)),
                pl.BlockSpec((BLKM, 1), lambda b, be_r: (b, 0)),
            ],
            out_specs=pl.BlockSpec((BLKM, D), lambda b, be_r: (b, 0)),
        ),
        out_shape=jax.ShapeDtypeStruct((P, D), jnp.float32),
    )(be, xs, W_gate_up.astype(jnp.bfloat16), W_down.astype(jnp.bfloat16),
      row_weight.reshape(P, 1))

    g0, g1 = _sc_combine_gather()(d0.reshape(T), d1.reshape(T), outs)

    out = pl.pallas_call(
        _final_kernel,
        grid=(nt,),
        in_specs=[
            pl.BlockSpec((BT, D), lambda t: (t, 0)),
            pl.BlockSpec((BT, D), lambda t: (t, 0)),
            pl.BlockSpec((BT, D), lambda t: (t, 0)),
        ],
        out_specs=pl.BlockSpec((BT, D), lambda t: (t, 0)),
        out_shape=jax.ShapeDtypeStruct((T, D), jnp.float32),
    )(h1, g0, g1)

    return out


# speedup vs baseline: 1.0272x; 1.0272x over previous
import functools
import jax
import jax.numpy as jnp
from jax import lax
from jax.experimental import pallas as pl
from jax.experimental.pallas import tpu as pltpu
from jax.experimental.pallas import tpu_sc as plsc

T = 2048
D = 1024
H = 16
KVH = 4
HD = 64
E = 8
TOPK = 2
I = 768
THETA = 1000000.0
EPS = 1e-6

BT = 256
BK = 512
EPAD = 128
NEG = jnp.finfo(jnp.float32).min


def _norm_rope(xall, nheads, cs64, sn64s, wn_tiled):
    w = nheads * HD
    hm = (jax.lax.broadcasted_iota(jnp.int32, (w, nheads), 0) // HD
          == jax.lax.broadcasted_iota(jnp.int32, (w, nheads), 1)).astype(jnp.float32)
    hmt = (jax.lax.broadcasted_iota(jnp.int32, (nheads, w), 0)
           == jax.lax.broadcasted_iota(jnp.int32, (nheads, w), 1) // HD).astype(jnp.float32)
    ssq = jnp.dot(xall * xall, hm, preferred_element_type=jnp.float32)
    invn = jax.lax.rsqrt(ssq * (1.0 / HD) + EPS)
    invb = jnp.dot(invn, hmt, preferred_element_type=jnp.float32)
    xn = xall * invb * wn_tiled
    lane = jax.lax.broadcasted_iota(jnp.int32, (BT, w), 1)
    first_half = (lane % HD) < (HD // 2)
    rot = jnp.where(first_half,
                    pltpu.roll(xn, w - HD // 2, axis=1),
                    pltpu.roll(xn, HD // 2, axis=1))
    csf = jnp.concatenate([cs64] * nheads, axis=1)
    snf = jnp.concatenate([sn64s] * nheads, axis=1)
    return xn * csf + rot * snf


def _pre_kernel(pos_ref, x_ref, w_ref, ln1_ref, qn_ref, kn_ref, q_ref, k_ref, v_ref):
    x = x_ref[...]
    h = x * jax.lax.rsqrt(jnp.mean(x * x, axis=-1, keepdims=True) + EPS) * ln1_ref[...]
    qkv = jnp.dot(h.astype(jnp.bfloat16), w_ref[...], preferred_element_type=jnp.float32)
    pos = pos_ref[...].astype(jnp.float32)
    half_iota = jax.lax.broadcasted_iota(jnp.int32, (1, HD // 2), 1).astype(jnp.float32)
    inv_freq = jnp.exp(half_iota * (-2.0 / HD) * jnp.log(THETA))
    freqs = pos * inv_freq
    cs = jnp.cos(freqs)
    sn = jnp.sin(freqs)
    cs64 = jnp.concatenate([cs, cs], axis=1)
    sn64s = jnp.concatenate([-sn, sn], axis=1)
    q_rope = _norm_rope(qkv[:, :H * HD], H, cs64, sn64s, qn_ref[...]).astype(jnp.bfloat16)
    k_rope = _norm_rope(qkv[:, H * HD:(H + KVH) * HD], KVH, cs64, sn64s,
                        kn_ref[...]).astype(jnp.bfloat16)
    for hh in range(H):
        q_ref[hh] = q_rope[:, hh * HD:(hh + 1) * HD]
    for hh in range(KVH):
        k_ref[hh] = k_rope[:, hh * HD:(hh + 1) * HD]
        v_ref[hh] = qkv[:, (H + KVH) * HD + hh * HD:
                        (H + KVH) * HD + (hh + 1) * HD].astype(jnp.bfloat16)


REP = H // KVH


def _flash_kernel(q_ref, k_ref, v_ref, o_ref):
    tq = pl.program_id(1)
    qs = [q_ref[i] * jnp.bfloat16(HD ** -0.5) for i in range(REP)]

    def chunk(j, ms, ls, accs, masked):
        k = k_ref[0, pl.ds(j * BK, BK), :]
        v = v_ref[0, pl.ds(j * BK, BK), :]
        if masked:
            rows = tq * BT + jax.lax.broadcasted_iota(jnp.int32, (BT, BK), 0)
            cols = j * BK + jax.lax.broadcasted_iota(jnp.int32, (BT, BK), 1)
            keep = rows >= cols
        nms, nls, naccs = [], [], []
        for i in range(REP):
            s = jax.lax.dot_general(qs[i], k, (((1,), (1,)), ((), ())),
                                    preferred_element_type=jnp.float32)
            if masked:
                s = jnp.where(keep, s, NEG)
            m_new = jnp.maximum(ms[i], jnp.max(s, axis=-1, keepdims=True))
            p = jnp.exp(s - m_new)
            alpha = jnp.exp(ms[i] - m_new)
            nls.append(ls[i] * alpha + jnp.sum(p, axis=-1, keepdims=True))
            naccs.append(accs[i] * alpha
                         + jnp.dot(p.astype(jnp.bfloat16), v,
                                   preferred_element_type=jnp.float32))
            nms.append(m_new)
        return nms, nls, naccs

    m0 = [jnp.full((BT, 1), NEG, jnp.float32)] * REP
    l0 = [jnp.zeros((BT, 1), jnp.float32)] * REP
    a0 = [jnp.zeros((BT, HD), jnp.float32)] * REP
    nfull = (tq * BT) // BK
    ms, ls, accs = jax.lax.fori_loop(
        0, nfull, lambda j, c: chunk(j, *c, masked=False), (m0, l0, a0))
    ms, ls, accs = chunk(nfull, ms, ls, accs, masked=True)
    for i in range(REP):
        o_ref[i] = (accs[i] / ls[i]).astype(jnp.bfloat16)


def _post_kernel(o_ref, res_ref, wo_ref, ln2_ref, gate_ref, h1_ref, h2_ref, w_ref):
    o_all = jnp.concatenate([o_ref[hh] for hh in range(H)], axis=1)
    attn = jnp.dot(o_all, wo_ref[...], preferred_element_type=jnp.float32)
    h1 = res_ref[...] + attn
    h1_ref[...] = h1
    h2 = h1 * jax.lax.rsqrt(jnp.mean(h1 * h1, axis=-1, keepdims=True) + EPS) * ln2_ref[...]
    h2_ref[...] = h2
    logits = jnp.dot(h2, gate_ref[...], preferred_element_type=jnp.float32)
    col = jax.lax.broadcasted_iota(jnp.int32, (BT, EPAD), 1)
    valid = col < E
    lm = jnp.where(valid, logits, NEG)
    mx = jnp.max(lm, axis=-1, keepdims=True)
    p = jnp.where(valid, jnp.exp(lm - mx), 0.0)
    rw = p / jnp.sum(p, axis=-1, keepdims=True)
    m1 = jnp.max(rw, axis=-1, keepdims=True)
    i1 = jnp.min(jnp.where(rw == m1, col, EPAD), axis=-1, keepdims=True)
    f1 = col == i1
    rw2 = jnp.where(f1, -1.0, rw)
    m2 = jnp.max(rw2, axis=-1, keepdims=True)
    i2 = jnp.min(jnp.where(rw2 == m2, col, EPAD), axis=-1, keepdims=True)
    f2 = col == i2
    denom = m1 + m2
    w = (jnp.where(f1, m1, 0.0) + jnp.where(f2, m2, 0.0)) / denom
    w_ref[...] = w


BLKM = 256
P = 6144
NBLK = P // BLKM


def _sched_kernel(w_ref, dest_ref, w8_ref, be_ref, d0_ref, d1_ref):
    w = w_ref[...]
    maskf = (w > 0.0).astype(jnp.float32)
    ri = jax.lax.broadcasted_iota(jnp.int32, (BT, BT), 0)
    ci = jax.lax.broadcasted_iota(jnp.int32, (BT, BT), 1)
    lstrict = (ri > ci).astype(jnp.float32)
    base = jnp.zeros((1, EPAD), jnp.float32)
    ranks = []
    for c in range(T // BT):
        seg = maskf[c * BT:(c + 1) * BT]
        within = jnp.dot(lstrict, seg, preferred_element_type=jnp.float32)
        ranks.append(within + base)
        base = base + jnp.sum(seg, axis=0, keepdims=True)
    rank = jnp.concatenate(ranks, axis=0)
    counts = base
    pc = jnp.ceil(counts * (1.0 / BLKM)) * BLKM
    ri2 = jax.lax.broadcasted_iota(jnp.int32, (EPAD, EPAD), 0)
    ci2 = jax.lax.broadcasted_iota(jnp.int32, (EPAD, EPAD), 1)
    ustrict = (ri2 < ci2).astype(jnp.float32)
    starts = jnp.dot(pc, ustrict, preferred_element_type=jnp.float32)
    dest = starts + rank
    dest_ref[...] = dest.astype(jnp.int32)
    w8_ref[...] = w
    bvals = (jax.lax.broadcasted_iota(jnp.int32, (EPAD, EPAD), 0) * BLKM).astype(jnp.float32)
    m = (jnp.broadcast_to(starts, (EPAD, EPAD)) <= bvals).astype(jnp.float32)
    be = jnp.sum(m, axis=1, keepdims=True) - 1.0
    be_ref[...] = jnp.minimum(be, float(E - 1)).astype(jnp.int32)
    mask = w > 0.0
    big = float(P - 1)
    d0 = jnp.min(jnp.where(mask, dest, big), axis=1, keepdims=True)
    d1 = jnp.min(jnp.where(mask & (dest > d0), dest, big), axis=1, keepdims=True)
    d0_ref[...] = d0.astype(jnp.int32)
    d1_ref[...] = d1.astype(jnp.int32)


def _group_kernel(be_ref, xs_ref, wgu_ref, wd_ref, ww_ref, out_ref):
    x = xs_ref[...].astype(jnp.bfloat16)
    gu = jnp.dot(x, wgu_ref[0], preferred_element_type=jnp.float32)
    g = gu[:, :I]
    u = gu[:, I:]
    act = (g / (1.0 + jnp.exp(-g))) * u * ww_ref[...]
    out_ref[...] = jnp.dot(act.astype(jnp.bfloat16), wd_ref[0],
                           preferred_element_type=jnp.float32)


def _final_kernel(h1_ref, g0_ref, g1_ref, out_ref):
    out_ref[...] = h1_ref[...] + g0_ref[...] + g1_ref[...]


SC_NC = 2
SC_NS = 16
SC_NW = SC_NC * SC_NS
LANES = 16


@functools.cache
def _sc_dispatch():
    @functools.partial(
        pl.kernel,
        mesh=plsc.VectorSubcoreMesh(core_axis_name="c", subcore_axis_name="s"),
        compiler_params=pltpu.CompilerParams(needs_layout_passes=False),
        out_type=jax.ShapeDtypeStruct((P,), jnp.float32),
        scratch_types=[
            pltpu.VMEM((T * E,), jnp.int32),
            pltpu.VMEM((T * E,), jnp.float32),
            pltpu.VMEM((P,), jnp.float32),
        ],
    )
    def dispatch(dest_hbm, w8_hbm, ww_out, dest_v, wv_v, ww_v):
        cid = lax.axis_index("c")
        sid = lax.axis_index("s")

        @pl.when((cid == 0) & (sid == 0))
        def _():
            pltpu.sync_copy(dest_hbm, dest_v)
            pltpu.sync_copy(w8_hbm, wv_v)

            def init(i, carry):
                ww_v[pl.ds(i * LANES, LANES)] = jnp.zeros((LANES,), jnp.float32)
                return carry

            lax.fori_loop(0, P // LANES, init, 0)

            def body(i, carry):
                d = dest_v[pl.ds(i * LANES, LANES)]
                wv = wv_v[pl.ds(i * LANES, LANES)]
                m = wv > 0.0
                plsc.store_scatter(ww_v, [d], wv, mask=m)
                return carry

            lax.fori_loop(0, (T * E) // LANES, body, 0)
            pltpu.sync_copy(ww_v, ww_out)

    return dispatch


SCH = T // SC_NW


@functools.cache
def _sc_scatter_x():
    @functools.partial(
        pl.kernel,
        mesh=plsc.VectorSubcoreMesh(core_axis_name="c", subcore_axis_name="s"),
        compiler_params=pltpu.CompilerParams(needs_layout_passes=False),
        out_type=jax.ShapeDtypeStruct((P, D), jnp.float32),
        scratch_types=[
            pltpu.VMEM((SCH,), jnp.int32),
            pltpu.VMEM((SCH,), jnp.int32),
            pltpu.VMEM((SCH, D), jnp.float32),
            pltpu.SemaphoreType.DMA,
            pltpu.SemaphoreType.DMA,
        ],
    )
    def scatter_x(h2_hbm, d0_hbm, d1_hbm, xs_out, idx0, idx1, rows_v, s0, s1):
        wid = lax.axis_index("s") * SC_NC + lax.axis_index("c")
        base = wid * SCH
        pltpu.sync_copy(h2_hbm.at[pl.ds(base, SCH)], rows_v)
        pltpu.sync_copy(d0_hbm.at[pl.ds(base, SCH)], idx0)
        pltpu.sync_copy(d1_hbm.at[pl.ds(base, SCH)], idx1)
        c0 = pltpu.async_copy(rows_v, xs_out.at[idx0], s0)
        c1 = pltpu.async_copy(rows_v, xs_out.at[idx1], s1)
        c0.wait()
        c1.wait()

    return scatter_x


CCH = 32


@functools.cache
def _sc_combine_gather():
    @functools.partial(
        pl.kernel,
        mesh=plsc.VectorSubcoreMesh(core_axis_name="c", subcore_axis_name="s"),
        compiler_params=pltpu.CompilerParams(needs_layout_passes=False),
        out_type=[
            jax.ShapeDtypeStruct((T, D), jnp.float32),
            jax.ShapeDtypeStruct((T, D), jnp.float32),
        ],
        scratch_types=[
            pltpu.VMEM((CCH,), jnp.int32),
            pltpu.VMEM((CCH,), jnp.int32),
            pltpu.VMEM((CCH, D), jnp.float32),
            pltpu.VMEM((CCH, D), jnp.float32),
            pltpu.SemaphoreType.DMA,
            pltpu.SemaphoreType.DMA,
            pltpu.SemaphoreType.DMA,
            pltpu.SemaphoreType.DMA,
        ],
    )
    def combine(d0_hbm, d1_hbm, outs_hbm, g0_out, g1_out, idx0, idx1,
                rows0, rows1, gs0, gs1, os0, os1):
        wid = lax.axis_index("s") * SC_NC + lax.axis_index("c")
        base0 = wid * (T // SC_NW)
        nc = T // SC_NW // CCH
        jobs = []
        for c in range(nc):
            jobs.append((d0_hbm, g0_out, base0 + c * CCH))
            jobs.append((d1_hbm, g1_out, base0 + c * CCH))
        idx = [idx0, idx1]
        rows = [rows0, rows1]
        gs = [gs0, gs1]
        os_ = [os0, os1]
        cps = [None, None]
        ocs = [None, None]
        dsts = [None, None]
        for j, (src_idx, dst, base) in enumerate(jobs):
            b = j & 1
            if ocs[b] is not None:
                ocs[b].wait()
                ocs[b] = None
            pltpu.sync_copy(src_idx.at[pl.ds(base, CCH)], idx[b])
            cps[b] = pltpu.async_copy(outs_hbm.at[idx[b]], rows[b], gs[b])
            dsts[b] = (dst, base)
            ob = 1 - b
            if cps[ob] is not None:
                cps[ob].wait()
                cps[ob] = None
                pdst, pbase = dsts[ob]
                ocs[ob] = pltpu.async_copy(
                    rows[ob], pdst.at[pl.ds(pbase, CCH)], os_[ob])
        lb = (len(jobs) - 1) & 1
        cps[lb].wait()
        pdst, pbase = dsts[lb]
        ocs[lb] = pltpu.async_copy(rows[lb], pdst.at[pl.ds(pbase, CCH)], os_[lb])
        for b in (0, 1):
            if ocs[b] is not None:
                ocs[b].wait()

    return combine


def kernel(hidden_states, positions, W_qkv, q_norm_w, k_norm_w, W_o, ln1_w, ln2_w,
           gate_w, W_gate_up, W_down):
    pos2 = positions.reshape(T, 1)
    ln1 = ln1_w.reshape(1, D)
    ln2 = ln2_w.reshape(1, D)
    qn = jnp.tile(q_norm_w, H).reshape(1, H * HD)
    kn = jnp.tile(k_norm_w, KVH).reshape(1, KVH * HD)
    gate_pad = jnp.concatenate([gate_w, jnp.zeros((D, EPAD - E), jnp.float32)], axis=1)

    nt = T // BT
    q, k, v = pl.pallas_call(
        _pre_kernel,
        grid=(nt,),
        in_specs=[
            pl.BlockSpec((BT, 1), lambda t: (t, 0)),
            pl.BlockSpec((BT, D), lambda t: (t, 0)),
            pl.BlockSpec((D, (H + 2 * KVH) * HD), lambda t: (0, 0)),
            pl.BlockSpec((1, D), lambda t: (0, 0)),
            pl.BlockSpec((1, H * HD), lambda t: (0, 0)),
            pl.BlockSpec((1, KVH * HD), lambda t: (0, 0)),
        ],
        out_specs=[
            pl.BlockSpec((H, BT, HD), lambda t: (0, t, 0)),
            pl.BlockSpec((KVH, BT, HD), lambda t: (0, t, 0)),
            pl.BlockSpec((KVH, BT, HD), lambda t: (0, t, 0)),
        ],
        out_shape=[
            jax.ShapeDtypeStruct((H, T, HD), jnp.bfloat16),
            jax.ShapeDtypeStruct((KVH, T, HD), jnp.bfloat16),
            jax.ShapeDtypeStruct((KVH, T, HD), jnp.bfloat16),
        ],
    )(pos2, hidden_states, W_qkv.astype(jnp.bfloat16), ln1, qn, kn)

    o = pl.pallas_call(
        _flash_kernel,
        grid=(KVH, nt),
        in_specs=[
            pl.BlockSpec((REP, BT, HD), lambda g, t: (g, t, 0)),
            pl.BlockSpec((1, T, HD), lambda g, t: (g, 0, 0)),
            pl.BlockSpec((1, T, HD), lambda g, t: (g, 0, 0)),
        ],
        out_specs=pl.BlockSpec((REP, BT, HD), lambda g, t: (g, t, 0)),
        out_shape=jax.ShapeDtypeStruct((H, T, HD), jnp.bfloat16),
    )(q, k, v)

    h1, h2, w = pl.pallas_call(
        _post_kernel,
        grid=(nt,),
        in_specs=[
            pl.BlockSpec((H, BT, HD), lambda t: (0, t, 0)),
            pl.BlockSpec((BT, D), lambda t: (t, 0)),
            pl.BlockSpec((H * HD, D), lambda t: (0, 0)),
            pl.BlockSpec((1, D), lambda t: (0, 0)),
            pl.BlockSpec((D, EPAD), lambda t: (0, 0)),
        ],
        out_specs=[
            pl.BlockSpec((BT, D), lambda t: (t, 0)),
            pl.BlockSpec((BT, D), lambda t: (t, 0)),
            pl.BlockSpec((BT, EPAD), lambda t: (t, 0)),
        ],
        out_shape=[
            jax.ShapeDtypeStruct((T, D), jnp.float32),
            jax.ShapeDtypeStruct((T, D), jnp.float32),
            jax.ShapeDtypeStruct((T, EPAD), jnp.float32),
        ],
    )(o, hidden_states, W_o.astype(jnp.bfloat16), ln2, gate_pad)

    dest, w8, be2, d0, d1 = pl.pallas_call(
        _sched_kernel,
        grid=(1,),
        in_specs=[pl.BlockSpec((T, EPAD), lambda i: (0, 0))],
        out_specs=[
            pl.BlockSpec((T, EPAD), lambda i: (0, 0)),
            pl.BlockSpec((T, EPAD), lambda i: (0, 0)),
            pl.BlockSpec((EPAD, 1), lambda i: (0, 0)),
            pl.BlockSpec((T, 1), lambda i: (0, 0)),
            pl.BlockSpec((T, 1), lambda i: (0, 0)),
        ],
        out_shape=[
            jax.ShapeDtypeStruct((T, EPAD), jnp.int32),
            jax.ShapeDtypeStruct((T, EPAD), jnp.float32),
            jax.ShapeDtypeStruct((EPAD, 1), jnp.int32),
            jax.ShapeDtypeStruct((T, 1), jnp.int32),
            jax.ShapeDtypeStruct((T, 1), jnp.int32),
        ],
    )(w)
    be = be2.reshape(EPAD)[:NBLK]

    dest8 = dest[:, :E].reshape(T * E)
    w8f = w8[:, :E].reshape(T * E)
    row_weight = _sc_dispatch()(dest8, w8f)

    xs = _sc_scatter_x()(h2, d0.reshape(T), d1.reshape(T))

    outs = pl.pallas_call(
        _group_kernel,
        grid_spec=pltpu.PrefetchScalarGridSpec(
            num_scalar_prefetch=1,
            grid=(NBLK,),
            in_specs=[
                pl.BlockSpec((BLKM, D), lambda b, be_r: (b, 0)),
                pl.BlockSpec((1, D, 2 * I), lambda b, be_r: (be_r[b], 0, 0)),
                pl.BlockSpec((1, I, D), lambda b, be_r: (be_r[b], 0, 0)),
                pl.BlockSpec((BLKM, 1), lambda b, be_r: (b, 0)),
            ],
            out_specs=pl.BlockSpec((BLKM, D), lambda b, be_r: (b, 0)),
        ),
        out_shape=jax.ShapeDtypeStruct((P, D), jnp.float32),
    )(be, xs, W_gate_up.astype(jnp.bfloat16), W_down.astype(jnp.bfloat16),
      row_weight.reshape(P, 1))

    g0, g1 = _sc_combine_gather()(d0.reshape(T), d1.reshape(T), outs)

    out = pl.pallas_call(
        _final_kernel,
        grid=(nt,),
        in_specs=[
            pl.BlockSpec((BT, D), lambda t: (t, 0)),
            pl.BlockSpec((BT, D), lambda t: (t, 0)),
            pl.BlockSpec((BT, D), lambda t: (t, 0)),
        ],
        out_specs=pl.BlockSpec((BT, D), lambda t: (t, 0)),
        out_shape=jax.ShapeDtypeStruct((T, D), jnp.float32),
    )(h1, g0, g1)

    return out
